# baseline (speedup 1.0000x reference)
import jax
import jax.numpy as jnp
from jax import lax
from jax.experimental import pallas as pl
from jax.experimental.pallas import tpu as pltpu
from jax.experimental.pallas import tpu_sc as plsc

B, S = 16384, 50
DIM = 32
NC, NS = 2, 16
NW = NC * NS
BW = B // NW
IDX_PER_XFER = 128
XFERS = BW // IDX_PER_XFER
L = 16


def _gather_body(idx_hbm, table_hbm, out_hbm, idx0, idx1, rows0, rows1, t0,
                 t1, gsem0, gsem1, wsem0, wsem1):
  wid = lax.axis_index("s") * NC + lax.axis_index("c")
  b0 = wid * BW

  def stage_and_fire(s, idx_v, rows_v, gsem):
    pltpu.sync_copy(idx_hbm.at[s, wid], idx_v)
    for j in range(XFERS):
      pltpu.async_copy(
          table_hbm.at[idx_v.at[j]],
          rows_v.at[pl.ds(j * IDX_PER_XFER, IDX_PER_XFER)],
          gsem,
      )

  def drain_gathers(idx_v, rows_v, gsem):
    for j in range(XFERS):
      pltpu.make_async_copy(
          table_hbm.at[idx_v.at[j]],
          rows_v.at[pl.ds(j * IDX_PER_XFER, IDX_PER_XFER)],
          gsem,
      ).wait()

  def transpose(rows_v, t_v):
    lanes = lax.iota(jnp.int32, L)

    def chunk(c4, carry):
      for cc in range(4):
        c = c4 * 4 + cc
        b_idx = c * L + lanes
        for f in range(DIM):
          f_idx = jnp.full((L,), f, jnp.int32)
          t_v[f, pl.ds(c * L, L)] = plsc.load_gather(rows_v, [b_idx, f_idx])
      return carry

    lax.fori_loop(0, BW // L // 4, chunk, 0)

  def write(s, t_v, wsem):
    pltpu.async_copy(t_v, out_hbm.at[s, :, pl.ds(b0, BW)], wsem)

  def drain_write(t_v, wsem):
    pltpu.make_async_copy(t_v, out_hbm.at[0, :, pl.ds(b0, BW)], wsem).wait()

  def step(k, first):
    for s_off, idx_v, rows_v, t_v, gsem, wsem in (
        (0, idx0, rows0, t0, gsem0, wsem0),
        (1, idx1, rows1, t1, gsem1, wsem1),
    ):
      s = k * 2 + s_off
      drain_gathers(idx_v, rows_v, gsem)
      if not first:
        drain_write(t_v, wsem)
      transpose(rows_v, t_v)

      @pl.when(k <= S // 2 - 2)
      def _():
        stage_and_fire(s + 2, idx_v, rows_v, gsem)

      write(s, t_v, wsem)

  stage_and_fire(0, idx0, rows0, gsem0)
  stage_and_fire(1, idx1, rows1, gsem1)
  step(0, True)
  lax.fori_loop(1, S // 2, lambda k, c: (step(k, False), c)[1], 0)
  drain_write(t0, wsem0)
  drain_write(t1, wsem1)


@jax.jit
def _embed(indices, table):
  idx4 = indices.T.reshape(S, NW, XFERS, IDX_PER_XFER).astype(jnp.int32)
  mesh = plsc.VectorSubcoreMesh(
      core_axis_name="c", subcore_axis_name="s", num_cores=NC, num_subcores=NS
  )
  out_t = pl.kernel(
      _gather_body,
      out_type=jax.ShapeDtypeStruct((S, DIM, B), jnp.float32),
      mesh=mesh,
      scratch_types=[
          pltpu.VMEM((XFERS, IDX_PER_XFER), jnp.int32),
          pltpu.VMEM((XFERS, IDX_PER_XFER), jnp.int32),
          pltpu.VMEM((BW, DIM), jnp.float32),
          pltpu.VMEM((BW, DIM), jnp.float32),
          pltpu.VMEM((DIM, BW), jnp.float32),
          pltpu.VMEM((DIM, BW), jnp.float32),
          pltpu.SemaphoreType.DMA,
          pltpu.SemaphoreType.DMA,
          pltpu.SemaphoreType.DMA,
          pltpu.SemaphoreType.DMA,
      ],
      compiler_params=pltpu.CompilerParams(
          use_tc_tiling_on_sc=False,
          needs_layout_passes=False,
          disable_bounds_checks=True,
      ),
  )(idx4, table)
  return out_t


def kernel(indices, table):
  out_t = _embed(indices, table)
  return out_t.transpose(2, 0, 1)


# speedup vs baseline: 1.3011x; 1.3011x over previous
import jax
import jax.numpy as jnp
from jax import lax
from jax.experimental import pallas as pl
from jax.experimental.pallas import tpu as pltpu
from jax.experimental.pallas import tpu_sc as plsc

B, S = 16384, 50
DIM = 32
NC, NS = 2, 16
NW = NC * NS
BW = B // NW
IDX_PER_XFER = 128
XFERS = BW // IDX_PER_XFER


def _gather_body(idx_hbm, table_hbm, out_hbm, idx0, idx1, rows0, rows1,
                 gsem0, gsem1, wsem0, wsem1):
  wid = lax.axis_index("s") * NC + lax.axis_index("c")
  b0 = wid * BW

  def stage_and_fire(s, idx_v, rows_v, gsem):
    pltpu.sync_copy(idx_hbm.at[s, wid], idx_v)
    for j in range(XFERS):
      pltpu.async_copy(
          table_hbm.at[idx_v.at[j]],
          rows_v.at[pl.ds(j * IDX_PER_XFER, IDX_PER_XFER)],
          gsem,
      )

  def drain_gathers(idx_v, rows_v, gsem):
    for j in range(XFERS):
      pltpu.make_async_copy(
          table_hbm.at[idx_v.at[j]],
          rows_v.at[pl.ds(j * IDX_PER_XFER, IDX_PER_XFER)],
          gsem,
      ).wait()

  def write(s, rows_v, wsem):
    pltpu.async_copy(rows_v, out_hbm.at[s, pl.ds(b0, BW)], wsem)

  def drain_write(rows_v, wsem):
    pltpu.make_async_copy(rows_v, out_hbm.at[0, pl.ds(b0, BW)], wsem).wait()

  def step(k, first):
    for s_off, idx_v, rows_v, gsem, wsem in (
        (0, idx0, rows0, gsem0, wsem0),
        (1, idx1, rows1, gsem1, wsem1),
    ):
      s = k * 2 + s_off
      drain_gathers(idx_v, rows_v, gsem)
      write(s, rows_v, wsem)

      @pl.when(k <= S // 2 - 2)
      def _():
        drain_write(rows_v, wsem)
        stage_and_fire(s + 2, idx_v, rows_v, gsem)

  stage_and_fire(0, idx0, rows0, gsem0)
  stage_and_fire(1, idx1, rows1, gsem1)
  step(0, True)
  lax.fori_loop(1, S // 2, lambda k, c: (step(k, False), c)[1], 0)
  drain_write(rows0, wsem0)
  drain_write(rows1, wsem1)


@jax.jit
def _embed(indices, table):
  idx4 = indices.T.reshape(S, NW, XFERS, IDX_PER_XFER).astype(jnp.int32)
  mesh = plsc.VectorSubcoreMesh(
      core_axis_name="c", subcore_axis_name="s", num_cores=NC, num_subcores=NS
  )
  out_t = pl.kernel(
      _gather_body,
      out_type=jax.ShapeDtypeStruct((S, B, DIM), jnp.float32),
      mesh=mesh,
      scratch_types=[
          pltpu.VMEM((XFERS, IDX_PER_XFER), jnp.int32),
          pltpu.VMEM((XFERS, IDX_PER_XFER), jnp.int32),
          pltpu.VMEM((BW, DIM), jnp.float32),
          pltpu.VMEM((BW, DIM), jnp.float32),
          pltpu.SemaphoreType.DMA,
          pltpu.SemaphoreType.DMA,
          pltpu.SemaphoreType.DMA,
          pltpu.SemaphoreType.DMA,
      ],
      compiler_params=pltpu.CompilerParams(
          use_tc_tiling_on_sc=False,
          needs_layout_passes=False,
          disable_bounds_checks=True,
      ),
  )(idx4, table)
  return out_t


def kernel(indices, table):
  out_t = _embed(indices, table)
  return out_t.transpose(1, 0, 2)
